# baseline (speedup 1.0000x reference)
import functools

import jax
import jax.numpy as jnp
from jax import lax
from jax.experimental import pallas as pl
from jax.experimental.pallas import tpu as pltpu
from jax.experimental.pallas import tpu_sc as plsc

_NC = 2
_NS = 16
_L = 16
_CHUNK = 20000
_NBUF = 4


def _sc_body(x_hbm, t_hbm, o_hbm, tgt_v, buf, res_v, sems, *, B, V):
    NW = _NC * _NS
    rows_w = B // NW
    nchunk = V // _CHUNK
    total = rows_w * nchunk

    c_id = lax.axis_index("c")
    s_id = lax.axis_index("s")
    wid = s_id * _NC + c_id
    base_row = wid * rows_w

    tcp = pltpu.make_async_copy(
        t_hbm.at[pl.ds(base_row, rows_w)], tgt_v, sems.at[_NBUF])
    tcp.start()
    tcp.wait()

    def copy(k, slot):
        row = base_row + k // nchunk
        col = (k % nchunk) * _CHUNK
        return pltpu.make_async_copy(
            x_hbm.at[row, pl.ds(col, _CHUNK)], buf.at[slot], sems.at[slot])

    for k in range(_NBUF - 1):
        copy(k, k).start()

    neg = jnp.full((_L,), -jnp.inf, jnp.float32)
    lanes = lax.iota(jnp.int32, _L)

    def chunk_step(k, slot, nxt_slot, carry):
        m_acc, t_acc, res_vec = carry
        j = k // nchunk
        c = k % nchunk
        copy(k, slot).wait()
        nxt = k + _NBUF - 1

        @pl.when(nxt < total)
        def _():
            copy(nxt, nxt_slot).start()

        chunk = buf.at[slot]

        first = jnp.full((_L,), c == 0)
        m_acc = jnp.where(first, neg, m_acc)
        t_acc = jnp.where(first, neg, t_acc)

        tvec = tgt_v[pl.ds((j // _L) * _L, _L)]
        t = jnp.max(jnp.where(lanes == lax.rem(j, _L), tvec, 0))
        loc = t - c * _CHUNK
        in_rng = (loc >= 0) & (loc < _CHUNK)
        loc_c = jnp.clip(loc, 0, _CHUNK - 1)
        base = (loc_c // _L) * _L
        sel = (lanes == loc_c - base) & jnp.full((_L,), in_rng)
        v = chunk[pl.ds(base, _L)]
        t_acc = jnp.where(sel, v, t_acc)
        chunk[pl.ds(base, _L)] = jnp.where(sel, neg, v)

        def vmax_body(i, accs):
            a0, a1, a2, a3 = accs
            b0 = chunk[pl.ds(i * 64, _L)]
            b1 = chunk[pl.ds(i * 64 + 16, _L)]
            b2 = chunk[pl.ds(i * 64 + 32, _L)]
            b3 = chunk[pl.ds(i * 64 + 48, _L)]
            return (jnp.maximum(a0, b0), jnp.maximum(a1, b1),
                    jnp.maximum(a2, b2), jnp.maximum(a3, b3))

        a0, a1, a2, a3 = lax.fori_loop(
            0, _CHUNK // 64, vmax_body, (m_acc, neg, neg, neg))
        for off in range((_CHUNK // 64) * 64, _CHUNK, _L):
            a0 = jnp.maximum(a0, chunk[pl.ds(off, _L)])
        m_acc = jnp.maximum(jnp.maximum(a0, a1), jnp.maximum(a2, a3))

        done = c == nchunk - 1
        r = jnp.max(m_acc) - jnp.max(t_acc)
        res_vec = jnp.where((lanes == lax.rem(j, _L)) & jnp.full((_L,), done),
                            r, res_vec)

        @pl.when(done & (lax.rem(j, _L) == _L - 1))
        def _():
            res_v[pl.ds((j // _L) * _L, _L)] = res_vec

        return (m_acc, t_acc, res_vec)

    def group_step(gr, carry):
        for b in range(_NBUF):
            carry = chunk_step(gr * _NBUF + b, b, (b + _NBUF - 1) % _NBUF,
                               carry)
        return carry

    lax.fori_loop(0, total // _NBUF, group_step, (neg, neg, neg))

    ocp = pltpu.make_async_copy(
        res_v, o_hbm.at[pl.ds(base_row, rows_w)], sems.at[_NBUF])
    ocp.start()
    ocp.wait()


def kernel(logits, target):
    B, V = logits.shape
    NW = _NC * _NS
    rows_w = B // NW
    t32 = target.astype(jnp.int32)
    mesh = plsc.VectorSubcoreMesh(core_axis_name="c", subcore_axis_name="s")
    run = pl.kernel(
        functools.partial(_sc_body, B=B, V=V),
        out_type=jax.ShapeDtypeStruct((B,), jnp.float32),
        mesh=mesh,
        compiler_params=pltpu.CompilerParams(
            use_tc_tiling_on_sc=False, needs_layout_passes=False),
        scratch_types=[
            pltpu.VMEM((rows_w,), jnp.int32),
            pltpu.VMEM((_NBUF, _CHUNK), jnp.float32),
            pltpu.VMEM((rows_w,), jnp.float32),
            pltpu.SemaphoreType.DMA((_NBUF + 1,)),
        ],
    )
    return run(logits, t32)


# speedup vs baseline: 2.0563x; 2.0563x over previous
import functools

import jax
import jax.numpy as jnp
from jax import lax
from jax.experimental import pallas as pl
from jax.experimental.pallas import tpu as pltpu
from jax.experimental.pallas import tpu_sc as plsc

_NC = 2
_NS = 16
_L = 16
_RG = 8
_CW = 1408
_NBUF = 4


def _sc_body(x_hbm, t_hbm, o_hbm, tgt_v, buf, tailb, res_v, sems, *, B, V):
    NW = _NC * _NS
    rows_w = B // NW
    groups_w = rows_w // _RG
    vfull = (V // 128) * 128
    nchunk = vfull // _CW
    tail_w = V - vfull

    c_id = lax.axis_index("c")
    s_id = lax.axis_index("s")
    wid = s_id * _NC + c_id
    base_row = wid * rows_w

    tcp = pltpu.make_async_copy(
        t_hbm.at[pl.ds(base_row, rows_w)], tgt_v, sems.at[_NBUF])
    tcp.start()
    tcp.wait()

    neg = jnp.full((_L,), -jnp.inf, jnp.float32)
    lanes = lax.iota(jnp.int32, _L)

    def group_body(g, res_vec):
        row0 = base_row + g * _RG

        def copy(c, slot):
            return pltpu.make_async_copy(
                x_hbm.at[pl.ds(row0, _RG), pl.ds(c * _CW, _CW)],
                buf.at[slot], sems.at[slot])

        for c in range(_NBUF - 1):
            copy(c, c).start()

        ts = []
        for r in range(_RG):
            j = g * _RG + r
            tvec = tgt_v[pl.ds((j // _L) * _L, _L)]
            ts.append(jnp.max(jnp.where(lanes == lax.rem(j, _L), tvec, 0)))

        def chunk_compute(chunk, width, col0, carry):
            accs = list(carry)
            for r in range(_RG):
                loc = ts[r] - col0
                in_rng = (loc >= 0) & (loc < width)
                loc_c = jnp.clip(loc, 0, width - 1)
                b0 = (loc_c // _L) * _L
                sel = (lanes == loc_c - b0) & jnp.full((_L,), in_rng)
                v = chunk[r, pl.ds(b0, _L)]
                accs[_RG + r] = jnp.where(sel, v, accs[_RG + r])
                chunk[r, pl.ds(b0, _L)] = jnp.where(sel, neg, v)

            def vmax_body(i, a):
                a = list(a)
                for r in range(_RG):
                    a[r] = jnp.maximum(a[r], chunk[r, pl.ds(i * _L, _L)])
                return tuple(a)

            return lax.fori_loop(0, width // _L, vmax_body, tuple(accs))

        def chunk_step(c, slot, nxt_slot, start_next, carry):
            copy(c, slot).wait()
            if start_next:
                copy(c + _NBUF - 1, nxt_slot).start()
            return chunk_compute(buf.at[slot], _CW, c * _CW, carry)

        def ring_body(q, carry):
            for b in range(_NBUF):
                carry = chunk_step(q * _NBUF + b, b, (b + _NBUF - 1) % _NBUF,
                                   True, carry)
            return carry

        init = tuple([neg] * (2 * _RG))
        nfull = (nchunk // _NBUF) * _NBUF
        accs = lax.fori_loop(0, nchunk // _NBUF, ring_body, init)
        for c in range(nfull, nchunk):
            accs = chunk_step(c, c % _NBUF, None, False, accs)

        tc2 = pltpu.make_async_copy(
            x_hbm.at[pl.ds(row0, _RG), pl.ds(vfull, tail_w)],
            tailb, sems.at[_NBUF])
        tc2.start()
        tc2.wait()
        accs = chunk_compute(tailb, tail_w, vfull, accs)

        for r in range(_RG):
            val = jnp.max(accs[r]) - jnp.max(accs[_RG + r])
            lane = lax.rem(g * _RG + r, _L)
            res_vec = jnp.where(lanes == lane, val, res_vec)

        @pl.when(lax.rem(g, 2) == 1)
        def _():
            res_v[pl.ds((g // 2) * _L, _L)] = res_vec

        return res_vec

    lax.fori_loop(0, groups_w, group_body, neg)

    ocp = pltpu.make_async_copy(
        res_v, o_hbm.at[pl.ds(base_row, rows_w)], sems.at[_NBUF])
    ocp.start()
    ocp.wait()


def kernel(logits, target):
    B, V = logits.shape
    NW = _NC * _NS
    rows_w = B // NW
    t32 = target.astype(jnp.int32)
    mesh = plsc.VectorSubcoreMesh(core_axis_name="c", subcore_axis_name="s")
    run = pl.kernel(
        functools.partial(_sc_body, B=B, V=V),
        out_type=jax.ShapeDtypeStruct((B,), jnp.float32),
        mesh=mesh,
        compiler_params=pltpu.CompilerParams(needs_layout_passes=False),
        scratch_types=[
            pltpu.VMEM((rows_w,), jnp.int32),
            pltpu.VMEM((_NBUF, _RG, _CW), jnp.float32),
            pltpu.VMEM((_RG, V - (V // 128) * 128), jnp.float32),
            pltpu.VMEM((rows_w,), jnp.float32),
            pltpu.SemaphoreType.DMA((_NBUF + 1,)),
        ],
    )
    return run(logits, t32)


# speedup vs baseline: 8.0330x; 3.9065x over previous
import functools

import jax
import jax.numpy as jnp
from jax.experimental import pallas as pl
from jax.experimental.pallas import tpu as pltpu

_BS = 2000


def _body(t_ref, x_ref, o_ref, macc, tacc, *, BS, V, B):
    i = pl.program_id(0)

    @pl.when(i == 0)
    def _():
        macc[...] = jnp.full((1, B), -jnp.inf, jnp.float32)
        tacc[...] = jnp.full((1, B), -jnp.inf, jnp.float32)

    x = x_ref[...]
    t = t_ref[...]
    idx = jax.lax.broadcasted_iota(jnp.int32, (BS, B), 0) + i * BS
    eq = idx == t
    neg = jnp.float32(-jnp.inf)
    mpart = jnp.max(jnp.where(eq, neg, x), axis=0, keepdims=True)
    tpart = jnp.max(jnp.where(eq, x, neg), axis=0, keepdims=True)
    macc[...] = jnp.maximum(macc[...], mpart)
    tacc[...] = jnp.maximum(tacc[...], tpart)

    @pl.when(i == pl.num_programs(0) - 1)
    def _():
        o_ref[...] = macc[...] - tacc[...]


def kernel(logits, target):
    B, V = logits.shape
    xt = logits.T
    t2 = target.astype(jnp.int32).reshape(1, B)
    BS = _BS
    out = pl.pallas_call(
        functools.partial(_body, BS=BS, V=V, B=B),
        grid=(V // BS,),
        in_specs=[
            pl.BlockSpec((1, B), lambda i: (0, 0)),
            pl.BlockSpec((BS, B), lambda i: (i, 0)),
        ],
        out_specs=pl.BlockSpec((1, B), lambda i: (0, 0)),
        out_shape=jax.ShapeDtypeStruct((1, B), jnp.float32),
        scratch_shapes=[
            pltpu.VMEM((1, B), jnp.float32),
            pltpu.VMEM((1, B), jnp.float32),
        ],
    )(t2, xt)
    return out.reshape(B)
